# baseline (speedup 1.0000x reference)
import functools

import jax
import jax.numpy as jnp
from jax import lax
from jax.experimental import pallas as pl
from jax.experimental.pallas import tpu as pltpu
from jax.experimental.pallas import tpu_sc as plsc

DIM = 32
NC, NS = 2, 16
NW = NC * NS
BATCH_PER_W = 512
STEP_BATCH = 64
SUB_BATCH = 16


def _tc_project_wide(tableT, w128):
    m = tableT.shape[1]
    bn = 8192
    grid = pl.cdiv(m, bn)

    def body(x_ref, w_ref, o_ref):
        o_ref[...] = lax.dot_general(
            x_ref[...], w_ref[...], (((0,), (0,)), ((), ())),
            preferred_element_type=jnp.float32)

    return pl.pallas_call(
        body,
        grid=(grid,),
        in_specs=[pl.BlockSpec((DIM, bn), lambda i: (0, i)),
                  pl.BlockSpec((DIM, 128), lambda i: (0, 0))],
        out_specs=pl.BlockSpec((bn, 128), lambda i: (i, 0)),
        out_shape=jax.ShapeDtypeStruct((m, 128), jnp.float32),
    )(tableT, w128)


def _sc_gather_t(ptable, indexes):
    bsz, f = indexes.shape
    fj = f * DIM
    bcols = bsz // STEP_BATCH
    mesh = plsc.VectorSubcoreMesh(core_axis_name="c", subcore_axis_name="s")
    steps = BATCH_PER_W // STEP_BATCH
    subs = STEP_BATCH // SUB_BATCH
    xput = 104
    nxf = fj // xput

    @functools.partial(
        pl.kernel,
        mesh=mesh,
        compiler_params=pltpu.CompilerParams(use_tc_tiling_on_sc=False,
                                             needs_layout_passes=False),
        out_type=jax.ShapeDtypeStruct((fj * bcols, STEP_BATCH), jnp.float32),
        scratch_types=[
            pltpu.VMEM((BATCH_PER_W, f), jnp.int32),
            pltpu.VMEM((SUB_BATCH * f, 128), jnp.float32),
            pltpu.VMEM((fj, STEP_BATCH), jnp.float32),
            pltpu.VMEM((nxf, 112), jnp.int32),
            pltpu.SemaphoreType.DMA,
            pltpu.SemaphoreType.DMA,
        ],
    )
    def k(table_hbm, idx_hbm, out_hbm, idx_v, rows_v, tr_v, sidx_v, sem, sem2):
        wid = lax.axis_index("s") * NC + lax.axis_index("c")
        batch0 = wid * BATCH_PER_W
        pltpu.sync_copy(idx_hbm.at[pl.ds(batch0, BATCH_PER_W)], idx_v)
        iota = lax.iota(jnp.int32, 16)
        rowsel = iota * f

        def sub(g, carry):
            s = g // subs
            c = g % subs
            coff = c * SUB_BATCH
            copies = []
            for t in range(SUB_BATCH):
                copies.append(pltpu.async_copy(
                    table_hbm.at[idx_v.at[g * SUB_BATCH + t]],
                    rows_v.at[pl.ds(t * f, f)],
                    sem,
                ))
            for cp in copies:
                cp.wait()

            for fi in range(f):
                rvec = rowsel + fi
                for j in range(DIM):
                    vals = plsc.load_gather(
                        rows_v, [rvec, jnp.full((16,), j, jnp.int32)])
                    tr_v[fi * DIM + j, pl.ds(coff, SUB_BATCH)] = vals

            @pl.when(c == subs - 1)
            def _scatter():
                bc = wid * steps + s
                scats = []
                for t in range(nxf):
                    for kk in range(7):
                        ids = (iota + (t * xput + 16 * kk)) * bcols + bc
                        sidx_v[t, pl.ds(16 * kk, 16)] = ids
                    scats.append(pltpu.async_copy(
                        tr_v.at[pl.ds(t * xput, xput)],
                        out_hbm.at[sidx_v.at[t, pl.ds(0, xput)]],
                        sem2,
                    ))
                for sc in scats:
                    sc.wait()

            return carry

        lax.fori_loop(0, steps * subs, sub, 0)

    return k(ptable, indexes)


def kernel(indexes, table, W):
    b, f = indexes.shape
    idx = indexes.astype(jnp.int32)
    w128 = jnp.pad(W.T, ((0, 0), (0, 96)))
    ptable = _tc_project_wide(table.T, w128)
    outT = _sc_gather_t(ptable, idx)
    return outT.reshape(f, DIM, b).transpose(2, 0, 1)


# speedup vs baseline: 1.2476x; 1.2476x over previous
import functools

import jax
import jax.numpy as jnp
from jax import lax
from jax.experimental import pallas as pl
from jax.experimental.pallas import tpu as pltpu
from jax.experimental.pallas import tpu_sc as plsc

DIM = 32
NC, NS = 2, 16
NW = NC * NS
BATCH_PER_W = 512
STEP_BATCH = 16


def _tc_project_wide(tableT, w128):
    m = tableT.shape[1]
    bn = 8192
    grid = pl.cdiv(m, bn)

    def body(x_ref, w_ref, o_ref):
        o_ref[...] = lax.dot_general(
            x_ref[...], w_ref[...], (((0,), (0,)), ((), ())),
            preferred_element_type=jnp.float32)

    return pl.pallas_call(
        body,
        grid=(grid,),
        in_specs=[pl.BlockSpec((DIM, bn), lambda i: (0, i)),
                  pl.BlockSpec((DIM, 128), lambda i: (0, 0))],
        out_specs=pl.BlockSpec((bn, 128), lambda i: (i, 0)),
        out_shape=jax.ShapeDtypeStruct((m, 128), jnp.float32),
    )(tableT, w128)


def _sc_gather(ptable, indexes):
    bsz, f = indexes.shape
    mesh = plsc.VectorSubcoreMesh(core_axis_name="c", subcore_axis_name="s")
    steps = BATCH_PER_W // STEP_BATCH

    @functools.partial(
        pl.kernel,
        mesh=mesh,
        compiler_params=pltpu.CompilerParams(use_tc_tiling_on_sc=False),
        out_type=jax.ShapeDtypeStruct((bsz, f, DIM), jnp.float32),
        scratch_types=[
            pltpu.VMEM((BATCH_PER_W, f), jnp.int32),
            pltpu.VMEM((STEP_BATCH, f, 128), jnp.float32),
            pltpu.VMEM((STEP_BATCH, f, 128), jnp.float32),
            pltpu.SemaphoreType.DMA,
            pltpu.SemaphoreType.DMA,
        ],
    )
    def k(table_hbm, idx_hbm, out_hbm, idx_v, rows_a, rows_b, sem_a, sem_b):
        wid = lax.axis_index("s") * NC + lax.axis_index("c")
        batch0 = wid * BATCH_PER_W
        pltpu.sync_copy(idx_hbm.at[pl.ds(batch0, BATCH_PER_W)], idx_v)

        def fire(s, buf, sem):
            copies = []
            for t in range(STEP_BATCH):
                copies.append(pltpu.async_copy(
                    table_hbm.at[idx_v.at[s * STEP_BATCH + t]],
                    buf.at[t],
                    sem,
                ))
            return copies

        def drain_out(s, buf, copies):
            for c in copies:
                c.wait()
            pltpu.sync_copy(
                buf.at[:, :, pl.ds(0, DIM)],
                out_hbm.at[pl.ds(batch0 + s * STEP_BATCH, STEP_BATCH)])

        cps0 = fire(0, rows_a, sem_a)

        def pair(m, carry):
            s0 = m * 2
            cps1 = fire(s0 + 1, rows_b, sem_b)
            drain_out(s0, rows_a, cps0)

            @pl.when(m < steps // 2 - 1)
            def _refire():
                fire(s0 + 2, rows_a, sem_a)

            drain_out(s0 + 1, rows_b, cps1)
            return carry

        lax.fori_loop(0, steps // 2, pair, 0)

    return k(ptable, indexes)


def kernel(indexes, table, W):
    idx = indexes.astype(jnp.int32)
    w128 = jnp.pad(W.T, ((0, 0), (0, 96)))
    ptable = _tc_project_wide(table.T, w128)
    return _sc_gather(ptable, idx)


# speedup vs baseline: 1.3244x; 1.0615x over previous
import functools

import jax
import jax.numpy as jnp
from jax import lax
from jax.experimental import pallas as pl
from jax.experimental.pallas import tpu as pltpu
from jax.experimental.pallas import tpu_sc as plsc

DIM = 32
NC, NS = 2, 16
NW = NC * NS
BATCH_PER_W = 512
STEP_BATCH = 16


def _tc_project_wide(tableT, w128):
    m = tableT.shape[1]
    bn = 16384
    grid = pl.cdiv(m, bn)

    def body(x_ref, w_ref, o_ref):
        o_ref[...] = lax.dot_general(
            x_ref[...], w_ref[...], (((0,), (0,)), ((), ())),
            preferred_element_type=jnp.float32)

    return pl.pallas_call(
        body,
        grid=(grid,),
        in_specs=[pl.BlockSpec((DIM, bn), lambda i: (0, i)),
                  pl.BlockSpec((DIM, 128), lambda i: (0, 0))],
        out_specs=pl.BlockSpec((bn, 128), lambda i: (i, 0)),
        out_shape=jax.ShapeDtypeStruct((m, 128), jnp.float32),
    )(tableT, w128)


def _sc_gather(ptable, indexes):
    bsz, f = indexes.shape
    mesh = plsc.VectorSubcoreMesh(core_axis_name="c", subcore_axis_name="s")
    steps = BATCH_PER_W // STEP_BATCH

    @functools.partial(
        pl.kernel,
        mesh=mesh,
        compiler_params=pltpu.CompilerParams(use_tc_tiling_on_sc=False),
        out_type=jax.ShapeDtypeStruct((bsz, f, DIM), jnp.float32),
        scratch_types=[
            pltpu.VMEM((BATCH_PER_W, f), jnp.int32),
            pltpu.VMEM((STEP_BATCH, f, 128), jnp.float32),
            pltpu.VMEM((STEP_BATCH, f, 128), jnp.float32),
            pltpu.SemaphoreType.DMA,
            pltpu.SemaphoreType.DMA,
        ],
    )
    def k(table_hbm, idx_hbm, out_hbm, idx_v, rows_a, rows_b, sem_a, sem_b):
        wid = lax.axis_index("s") * NC + lax.axis_index("c")
        batch0 = wid * BATCH_PER_W
        pltpu.sync_copy(idx_hbm.at[pl.ds(batch0, BATCH_PER_W)], idx_v)

        def fire(s, buf, sem):
            copies = []
            for t in range(STEP_BATCH):
                copies.append(pltpu.async_copy(
                    table_hbm.at[idx_v.at[s * STEP_BATCH + t]],
                    buf.at[t],
                    sem,
                ))
            return copies

        def drain_out(s, buf, copies):
            for c in copies:
                c.wait()
            pltpu.sync_copy(
                buf.at[:, :, pl.ds(0, DIM)],
                out_hbm.at[pl.ds(batch0 + s * STEP_BATCH, STEP_BATCH)])

        cps0 = fire(0, rows_a, sem_a)

        def pair(m, carry):
            s0 = m * 2
            cps1 = fire(s0 + 1, rows_b, sem_b)
            drain_out(s0, rows_a, cps0)

            @pl.when(m < steps // 2 - 1)
            def _refire():
                fire(s0 + 2, rows_a, sem_a)

            drain_out(s0 + 1, rows_b, cps1)
            return carry

        lax.fori_loop(0, steps // 2, pair, 0)

    return k(ptable, indexes)


def kernel(indexes, table, W):
    idx = indexes.astype(jnp.int32)
    w128 = jnp.pad(W.T, ((0, 0), (0, 96)))
    ptable = _tc_project_wide(table.T, w128)
    return _sc_gather(ptable, idx)


# speedup vs baseline: 1.3372x; 1.0097x over previous
import functools

import jax
import jax.numpy as jnp
from jax import lax
from jax.experimental import pallas as pl
from jax.experimental.pallas import tpu as pltpu
from jax.experimental.pallas import tpu_sc as plsc

DIM = 32
NC, NS = 2, 16
NW = NC * NS
BATCH_PER_W = 512
STEP_BATCH = 16


def _tc_project_wide(tableT, w128):
    m = tableT.shape[1]
    bn = 32768
    grid = pl.cdiv(m, bn)

    def body(x_ref, w_ref, o_ref):
        o_ref[...] = lax.dot_general(
            x_ref[...], w_ref[...], (((0,), (0,)), ((), ())),
            preferred_element_type=jnp.float32)

    return pl.pallas_call(
        body,
        grid=(grid,),
        in_specs=[pl.BlockSpec((DIM, bn), lambda i: (0, i)),
                  pl.BlockSpec((DIM, 128), lambda i: (0, 0))],
        out_specs=pl.BlockSpec((bn, 128), lambda i: (i, 0)),
        out_shape=jax.ShapeDtypeStruct((m, 128), jnp.float32),
    )(tableT, w128)


def _sc_gather(ptable, indexes):
    bsz, f = indexes.shape
    mesh = plsc.VectorSubcoreMesh(core_axis_name="c", subcore_axis_name="s")
    steps = BATCH_PER_W // STEP_BATCH

    @functools.partial(
        pl.kernel,
        mesh=mesh,
        compiler_params=pltpu.CompilerParams(use_tc_tiling_on_sc=False),
        out_type=jax.ShapeDtypeStruct((bsz, f, DIM), jnp.float32),
        scratch_types=[
            pltpu.VMEM((BATCH_PER_W, f), jnp.int32),
            pltpu.VMEM((STEP_BATCH, f, 128), jnp.float32),
            pltpu.VMEM((STEP_BATCH, f, 128), jnp.float32),
            pltpu.SemaphoreType.DMA,
            pltpu.SemaphoreType.DMA,
        ],
    )
    def k(table_hbm, idx_hbm, out_hbm, idx_v, rows_a, rows_b, sem_a, sem_b):
        wid = lax.axis_index("s") * NC + lax.axis_index("c")
        batch0 = wid * BATCH_PER_W
        pltpu.sync_copy(idx_hbm.at[pl.ds(batch0, BATCH_PER_W)], idx_v)

        def fire(s, buf, sem):
            copies = []
            for t in range(STEP_BATCH):
                copies.append(pltpu.async_copy(
                    table_hbm.at[idx_v.at[s * STEP_BATCH + t]],
                    buf.at[t],
                    sem,
                ))
            return copies

        def drain_out(s, buf, copies):
            for c in copies:
                c.wait()
            pltpu.sync_copy(
                buf.at[:, :, pl.ds(0, DIM)],
                out_hbm.at[pl.ds(batch0 + s * STEP_BATCH, STEP_BATCH)])

        cps0 = fire(0, rows_a, sem_a)

        def pair(m, carry):
            s0 = m * 2
            cps1 = fire(s0 + 1, rows_b, sem_b)
            drain_out(s0, rows_a, cps0)

            @pl.when(m < steps // 2 - 1)
            def _refire():
                fire(s0 + 2, rows_a, sem_a)

            drain_out(s0 + 1, rows_b, cps1)
            return carry

        lax.fori_loop(0, steps // 2, pair, 0)

    return k(ptable, indexes)


def kernel(indexes, table, W):
    idx = indexes.astype(jnp.int32)
    w128 = jnp.pad(W.T, ((0, 0), (0, 96)))
    ptable = _tc_project_wide(table.T, w128)
    return _sc_gather(ptable, idx)
